# baseline (speedup 1.0000x reference)
import functools

import jax
import jax.numpy as jnp
from jax import lax
from jax.experimental import pallas as pl
from jax.experimental.pallas import tpu as pltpu
from jax.experimental.pallas import tpu_sc as plsc

N = 100000
N_IN = 128
N_HID = 32
M = 2000

M2 = 2048
NT = 16
BLK = 10000
NH = N // 2
HBLKS = NH // BLK
CH = 3120
TAIL = NH - NT * CH


def _mlp_body(x_ref, w1_ref, b1_ref, w2_ref, b2_ref, y_ref):
    x = x_ref[...]
    h = jnp.dot(x, w1_ref[...], preferred_element_type=jnp.float32)
    h = h + b1_ref[...]
    h = h * jax.nn.sigmoid(h)
    y = jnp.dot(h, w2_ref[...], preferred_element_type=jnp.float32)
    y_ref[...] = y + b2_ref[...]


def _mlp_half(x, W1, b1, W2, b2, half):
    off = half * HBLKS
    return pl.pallas_call(
        _mlp_body,
        grid=(HBLKS,),
        in_specs=[
            pl.BlockSpec((BLK, N_IN), lambda i: (i + off, 0)),
            pl.BlockSpec((N_IN, N_HID), lambda i: (0, 0)),
            pl.BlockSpec((1, N_HID), lambda i: (0, 0)),
            pl.BlockSpec((N_HID, 1), lambda i: (0, 0)),
            pl.BlockSpec((1, 1), lambda i: (0, 0)),
        ],
        out_specs=pl.BlockSpec((BLK, 1), lambda i: (i, 0)),
        out_shape=jax.ShapeDtypeStruct((NH, 1), jnp.float32),
    )(x, W1, b1.reshape(1, N_HID), W2, b2.reshape(1, 1))


def _seg_half(n0, add_prev):
    def body(y_hbm, idx_hbm, *rest):
        if add_prev:
            prev_hbm, out_hbm, idx_v, val_v, tidx_v, tval_v, acc_v, row_v, \
                red_v, out_v, prev_v, shared = rest
        else:
            out_hbm, idx_v, val_v, tidx_v, tval_v, acc_v, row_v, \
                red_v, out_v, prev_v, shared = rest

        wid = lax.axis_index("s")
        is_last = wid == NT - 1
        base = n0 + wid * CH
        pltpu.sync_copy(idx_hbm.at[pl.ds(base, CH)], idx_v)
        pltpu.sync_copy(y_hbm.at[pl.ds(wid * CH, CH)], val_v)
        pltpu.sync_copy(idx_hbm.at[pl.ds(n0 + NT * CH, TAIL)], tidx_v)
        pltpu.sync_copy(y_hbm.at[pl.ds(NT * CH, TAIL)], tval_v)

        lo = idx_v[pl.ds(0, 16)][0]
        hi = jnp.where(is_last, tidx_v[pl.ds(TAIL - 16, 16)][15],
                       idx_v[pl.ds(CH - 16, 16)][15])
        g0 = lax.shift_right_logical(lo, 4)
        g1 = lax.shift_right_logical(hi, 4)
        trips = g1 - g0 + 1

        zeros16 = jnp.zeros((16,), jnp.float32)

        def zrow_body(c, carry):
            row_v[pl.ds(c * 16, 16)] = zeros16
            return carry

        lax.fori_loop(0, M2 // 16, zrow_body, 0)

        def zero_body(c, carry):
            for r in range(NT):
                acc_v[pl.ds(r * M2 + (g0 + c) * 16, 16)] = zeros16
            return carry

        lax.fori_loop(0, trips, zero_body, 0)

        lane_off = lax.iota(jnp.int32, 16) * M2

        def scat_body(i, carry):
            iv = idx_v[pl.ds(i * 16, 16)]
            vv = val_v[pl.ds(i * 16, 16)]
            plsc.addupdate_scatter(acc_v, [iv + lane_off], vv)
            return carry

        lax.fori_loop(0, CH // 16, scat_body, 0)

        @pl.when(is_last)
        def _tail():
            def tscat_body(i, carry):
                iv = tidx_v[pl.ds(i * 16, 16)]
                vv = tval_v[pl.ds(i * 16, 16)]
                plsc.addupdate_scatter(acc_v, [iv + lane_off], vv)
                return carry

            lax.fori_loop(0, TAIL // 16, tscat_body, 0)

        def red_body(c, carry):
            s = acc_v[pl.ds((g0 + c) * 16, 16)]
            for r in range(1, NT):
                s = s + acc_v[pl.ds(r * M2 + (g0 + c) * 16, 16)]
            row_v[pl.ds((g0 + c) * 16, 16)] = s
            return carry

        lax.fori_loop(0, trips, red_body, 0)

        pltpu.sync_copy(row_v, shared.at[pl.ds(wid * M2, M2)])
        plsc.subcore_barrier()

        for r in range(NT):
            pltpu.sync_copy(shared.at[pl.ds(r * M2 + wid * 128, 128)],
                            red_v.at[pl.ds(r * 128, 128)])
        if add_prev:
            pltpu.sync_copy(prev_hbm.at[pl.ds(wid * 128, 128)], prev_v)

        def fin_body(c, carry):
            s = red_v[pl.ds(c * 16, 16)]
            for r in range(1, NT):
                s = s + red_v[pl.ds(r * 128 + c * 16, 16)]
            if add_prev:
                s = s + prev_v[pl.ds(c * 16, 16)]
            out_v[pl.ds(c * 16, 16)] = s
            return carry

        lax.fori_loop(0, 128 // 16, fin_body, 0)

        pltpu.sync_copy(out_v, out_hbm.at[pl.ds(wid * 128, 128)])

    return body


@functools.cache
def _make_seg(n0, add_prev):
    return functools.partial(
        pl.kernel,
        mesh=plsc.VectorSubcoreMesh(core_axis_name="c", subcore_axis_name="s",
                                    num_cores=1),
        out_type=jax.ShapeDtypeStruct((M2,), jnp.float32),
        compiler_params=pltpu.CompilerParams(
            use_tc_tiling_on_sc=False, needs_layout_passes=False),
        scratch_types=[
            pltpu.VMEM((CH,), jnp.int32),
            pltpu.VMEM((CH,), jnp.float32),
            pltpu.VMEM((TAIL,), jnp.int32),
            pltpu.VMEM((TAIL,), jnp.float32),
            pltpu.VMEM((NT * M2,), jnp.float32),
            pltpu.VMEM((M2,), jnp.float32),
            pltpu.VMEM((NT * 128,), jnp.float32),
            pltpu.VMEM((128,), jnp.float32),
            pltpu.VMEM((128,), jnp.float32),
            pltpu.VMEM_SHARED((NT * M2,), jnp.float32),
        ],
    )(_seg_half(n0, add_prev))


def kernel(scalar_representation, idx_m, W1, b1, W2, b2):
    y0 = _mlp_half(scalar_representation, W1, b1, W2, b2, 0).reshape(NH)
    s0 = _make_seg(0, False)(y0, idx_m)
    y1 = _mlp_half(scalar_representation, W1, b1, W2, b2, 1).reshape(NH)
    s1 = _make_seg(NH, True)(y1, idx_m, s0)
    return s1[:M]


# speedup vs baseline: 1.0510x; 1.0510x over previous
import functools

import jax
import jax.numpy as jnp
from jax import lax
from jax.experimental import pallas as pl
from jax.experimental.pallas import tpu as pltpu
from jax.experimental.pallas import tpu_sc as plsc

N = 100000
N_IN = 128
N_HID = 32
M = 2000

M2 = 2048
NT = 16
CH = 6240
TAIL = N - NT * CH
BLK = 10000
GRID = N // BLK


def _mlp_body(x_ref, w1_ref, b1_ref, w2_ref, b2_ref, y_ref):
    x = x_ref[...]
    h = jnp.dot(x, w1_ref[...], preferred_element_type=jnp.float32)
    h = h + b1_ref[...]
    h = h * jax.nn.sigmoid(h)
    y = jnp.dot(h, w2_ref[...], preferred_element_type=jnp.float32)
    y_ref[...] = y + b2_ref[...]


def _mlp(x, W1, b1, W2, b2):
    return pl.pallas_call(
        _mlp_body,
        grid=(GRID,),
        in_specs=[
            pl.BlockSpec((BLK, N_IN), lambda i: (i, 0)),
            pl.BlockSpec((N_IN, N_HID), lambda i: (0, 0)),
            pl.BlockSpec((1, N_HID), lambda i: (0, 0)),
            pl.BlockSpec((N_HID, 1), lambda i: (0, 0)),
            pl.BlockSpec((1, 1), lambda i: (0, 0)),
        ],
        out_specs=pl.BlockSpec((BLK, 1), lambda i: (i, 0)),
        out_shape=jax.ShapeDtypeStruct((N, 1), jnp.float32),
    )(x, W1, b1.reshape(1, N_HID), W2, b2.reshape(1, 1))


def _seg_body(y_hbm, idx_hbm, out_hbm,
              idx_v, val_v, tidx_v, tval_v, acc_v, row_v, red_v, out_v,
              shared):
    wid = lax.axis_index("s")
    is_last = wid == NT - 1
    base = wid * CH
    pltpu.sync_copy(idx_hbm.at[pl.ds(base, CH)], idx_v)
    pltpu.sync_copy(y_hbm.at[pl.ds(base, CH)], val_v)
    pltpu.sync_copy(idx_hbm.at[pl.ds(NT * CH, TAIL)], tidx_v)
    pltpu.sync_copy(y_hbm.at[pl.ds(NT * CH, TAIL)], tval_v)

    lo = idx_v[pl.ds(0, 16)][0]
    hi = jnp.where(is_last, tidx_v[pl.ds(TAIL - 16, 16)][15],
                   idx_v[pl.ds(CH - 16, 16)][15])
    g0 = lax.shift_right_logical(lo, 4)
    g1 = lax.shift_right_logical(hi, 4)
    trips = g1 - g0 + 1

    zeros16 = jnp.zeros((16,), jnp.float32)

    def zrow_body(c, carry):
        row_v[pl.ds(c * 16, 16)] = zeros16
        return carry

    lax.fori_loop(0, M2 // 16, zrow_body, 0)

    def zero_body(c, carry):
        for r in range(NT):
            acc_v[pl.ds(r * M2 + (g0 + c) * 16, 16)] = zeros16
        return carry

    lax.fori_loop(0, trips, zero_body, 0)

    lane_off = lax.iota(jnp.int32, 16) * M2

    def scat_body(i, carry):
        iv = idx_v[pl.ds(i * 16, 16)]
        vv = val_v[pl.ds(i * 16, 16)]
        plsc.addupdate_scatter(acc_v, [iv + lane_off], vv)
        return carry

    lax.fori_loop(0, CH // 16, scat_body, 0)

    @pl.when(is_last)
    def _tail():
        def tscat_body(i, carry):
            iv = tidx_v[pl.ds(i * 16, 16)]
            vv = tval_v[pl.ds(i * 16, 16)]
            plsc.addupdate_scatter(acc_v, [iv + lane_off], vv)
            return carry

        lax.fori_loop(0, TAIL // 16, tscat_body, 0)

    def red_body(c, carry):
        s = acc_v[pl.ds((g0 + c) * 16, 16)]
        for r in range(1, NT):
            s = s + acc_v[pl.ds(r * M2 + (g0 + c) * 16, 16)]
        row_v[pl.ds((g0 + c) * 16, 16)] = s
        return carry

    lax.fori_loop(0, trips, red_body, 0)

    pltpu.sync_copy(row_v, shared.at[pl.ds(wid * M2, M2)])
    plsc.subcore_barrier()

    for r in range(NT):
        pltpu.sync_copy(shared.at[pl.ds(r * M2 + wid * 128, 128)],
                        red_v.at[pl.ds(r * 128, 128)])

    def fin_body(c, carry):
        s = red_v[pl.ds(c * 16, 16)]
        for r in range(1, NT):
            s = s + red_v[pl.ds(r * 128 + c * 16, 16)]
        out_v[pl.ds(c * 16, 16)] = s
        return carry

    lax.fori_loop(0, 128 // 16, fin_body, 0)

    @pl.when(jnp.logical_not(is_last))
    def _full_write():
        pltpu.sync_copy(out_v, out_hbm.at[pl.ds(wid * 128, 128)])

    @pl.when(is_last)
    def _last_write():
        pltpu.sync_copy(out_v.at[pl.ds(0, M - (NT - 1) * 128)],
                        out_hbm.at[pl.ds((NT - 1) * 128,
                                         M - (NT - 1) * 128)])


@functools.cache
def _make_seg():
    @functools.partial(
        pl.kernel,
        mesh=plsc.VectorSubcoreMesh(core_axis_name="c", subcore_axis_name="s",
                                    num_cores=1),
        out_type=jax.ShapeDtypeStruct((M,), jnp.float32),
        compiler_params=pltpu.CompilerParams(
            use_tc_tiling_on_sc=False, needs_layout_passes=False),
        scratch_types=[
            pltpu.VMEM((CH,), jnp.int32),
            pltpu.VMEM((CH,), jnp.float32),
            pltpu.VMEM((TAIL,), jnp.int32),
            pltpu.VMEM((TAIL,), jnp.float32),
            pltpu.VMEM((NT * M2,), jnp.float32),
            pltpu.VMEM((M2,), jnp.float32),
            pltpu.VMEM((NT * 128,), jnp.float32),
            pltpu.VMEM((128,), jnp.float32),
            pltpu.VMEM_SHARED((NT * M2,), jnp.float32),
        ],
    )
    def _seg(y_hbm, idx_hbm, out_hbm, *scratch):
        _seg_body(y_hbm, idx_hbm, out_hbm, *scratch)

    return _seg


def kernel(scalar_representation, idx_m, W1, b1, W2, b2):
    y = _mlp(scalar_representation, W1, b1, W2, b2).reshape(N)
    return _make_seg()(y, idx_m)


# speedup vs baseline: 1.0714x; 1.0194x over previous
import functools

import jax
import jax.numpy as jnp
from jax import lax
from jax.experimental import pallas as pl
from jax.experimental.pallas import tpu as pltpu
from jax.experimental.pallas import tpu_sc as plsc

N = 100000
N_IN = 128
N_HID = 32
M = 2000

M2 = 2048
NT = 16
CH = 6240
TAIL = N - NT * CH
BLK = 10000
GRID = N // BLK


def _mlp_body(x_ref, w1_ref, b1_ref, w2_ref, b2_ref, y_ref):
    x = x_ref[...]
    h = jnp.dot(x, w1_ref[...], preferred_element_type=jnp.float32)
    h = h + b1_ref[...]
    h = h * jax.nn.sigmoid(h)
    y = jnp.dot(h, w2_ref[...], preferred_element_type=jnp.float32)
    y_ref[...] = y + b2_ref[...]


def _mlp(x, W1, b1, W2, b2):
    return pl.pallas_call(
        _mlp_body,
        grid=(GRID,),
        in_specs=[
            pl.BlockSpec((BLK, N_IN), lambda i: (i, 0)),
            pl.BlockSpec((N_IN, N_HID), lambda i: (0, 0)),
            pl.BlockSpec((1, N_HID), lambda i: (0, 0)),
            pl.BlockSpec((N_HID, 1), lambda i: (0, 0)),
            pl.BlockSpec((1, 1), lambda i: (0, 0)),
        ],
        out_specs=pl.BlockSpec((BLK, 1), lambda i: (i, 0)),
        out_shape=jax.ShapeDtypeStruct((N, 1), jnp.float32),
    )(x, W1, b1.reshape(1, N_HID), W2, b2.reshape(1, 1))


def _seg_body(y_hbm, idx_hbm, out_hbm,
              idx_v, val_v, tidx_v, tval_v, acc_v, row_v, red_v, out_v,
              shared):
    wid = lax.axis_index("s")
    is_last = wid == NT - 1
    base = wid * CH
    pltpu.sync_copy(idx_hbm.at[pl.ds(base, CH)], idx_v)
    pltpu.sync_copy(y_hbm.at[pl.ds(base, CH)], val_v)
    pltpu.sync_copy(idx_hbm.at[pl.ds(NT * CH, TAIL)], tidx_v)
    pltpu.sync_copy(y_hbm.at[pl.ds(NT * CH, TAIL)], tval_v)

    lo = idx_v[pl.ds(0, 16)][0]
    hi = jnp.where(is_last, tidx_v[pl.ds(TAIL - 16, 16)][15],
                   idx_v[pl.ds(CH - 16, 16)][15])
    g0 = lax.shift_right_logical(lo, 4)
    g1 = lax.shift_right_logical(hi, 4)
    trips = g1 - g0 + 1

    zeros16 = jnp.zeros((16,), jnp.float32)

    def zrow_body(c, carry):
        row_v[pl.ds(c * 16, 16)] = zeros16
        return carry

    lax.fori_loop(0, M2 // 16, zrow_body, 0)

    def zero_body(c, carry):
        for r in range(NT):
            acc_v[pl.ds(r * M2 + (g0 + c) * 16, 16)] = zeros16
        return carry

    lax.fori_loop(0, trips, zero_body, 0)

    lane_off = lax.iota(jnp.int32, 16) * M2

    def scat_body(i, carry):
        for u in range(6):
            iv = idx_v[pl.ds((i * 6 + u) * 16, 16)]
            vv = val_v[pl.ds((i * 6 + u) * 16, 16)]
            plsc.addupdate_scatter(acc_v, [iv + lane_off], vv)
        return carry

    lax.fori_loop(0, CH // 96, scat_body, 0)

    @pl.when(is_last)
    def _tail():
        def tscat_body(i, carry):
            iv = tidx_v[pl.ds(i * 16, 16)]
            vv = tval_v[pl.ds(i * 16, 16)]
            plsc.addupdate_scatter(acc_v, [iv + lane_off], vv)
            return carry

        lax.fori_loop(0, TAIL // 16, tscat_body, 0)

    def red_body(c, carry):
        s = acc_v[pl.ds((g0 + c) * 16, 16)]
        for r in range(1, NT):
            s = s + acc_v[pl.ds(r * M2 + (g0 + c) * 16, 16)]
        row_v[pl.ds((g0 + c) * 16, 16)] = s
        return carry

    lax.fori_loop(0, trips, red_body, 0)

    pltpu.sync_copy(row_v, shared.at[wid])
    plsc.subcore_barrier()

    pltpu.sync_copy(shared.at[:, pl.ds(wid * 128, 128)], red_v)

    def fin_body(c, carry):
        s = red_v[0, pl.ds(c * 16, 16)]
        for r in range(1, NT):
            s = s + red_v[r, pl.ds(c * 16, 16)]
        out_v[pl.ds(c * 16, 16)] = s
        return carry

    lax.fori_loop(0, 128 // 16, fin_body, 0)

    @pl.when(jnp.logical_not(is_last))
    def _full_write():
        pltpu.sync_copy(out_v, out_hbm.at[pl.ds(wid * 128, 128)])

    @pl.when(is_last)
    def _last_write():
        pltpu.sync_copy(out_v.at[pl.ds(0, M - (NT - 1) * 128)],
                        out_hbm.at[pl.ds((NT - 1) * 128,
                                         M - (NT - 1) * 128)])


@functools.cache
def _make_seg():
    @functools.partial(
        pl.kernel,
        mesh=plsc.VectorSubcoreMesh(core_axis_name="c", subcore_axis_name="s",
                                    num_cores=1),
        out_type=jax.ShapeDtypeStruct((M,), jnp.float32),
        compiler_params=pltpu.CompilerParams(
            use_tc_tiling_on_sc=False, needs_layout_passes=False),
        scratch_types=[
            pltpu.VMEM((CH,), jnp.int32),
            pltpu.VMEM((CH,), jnp.float32),
            pltpu.VMEM((TAIL,), jnp.int32),
            pltpu.VMEM((TAIL,), jnp.float32),
            pltpu.VMEM((NT * M2,), jnp.float32),
            pltpu.VMEM((M2,), jnp.float32),
            pltpu.VMEM((NT, 128), jnp.float32),
            pltpu.VMEM((128,), jnp.float32),
            pltpu.VMEM_SHARED((NT, M2), jnp.float32),
        ],
    )
    def _seg(y_hbm, idx_hbm, out_hbm, *scratch):
        _seg_body(y_hbm, idx_hbm, out_hbm, *scratch)

    return _seg


def kernel(scalar_representation, idx_m, W1, b1, W2, b2):
    y = _mlp(scalar_representation, W1, b1, W2, b2).reshape(N)
    return _make_seg()(y, idx_m)


# speedup vs baseline: 1.1116x; 1.0375x over previous
import functools

import jax
import jax.numpy as jnp
from jax import lax
from jax.experimental import pallas as pl
from jax.experimental.pallas import tpu as pltpu
from jax.experimental.pallas import tpu_sc as plsc

N = 100000
N_IN = 128
N_HID = 32
M = 2000

M2 = 2048
SR = M2 + 1
NT = 16
CH = 6240
TAIL = N - NT * CH
BLK = 10000
GRID = N // BLK


def _mlp_body(x_ref, w1_ref, b1_ref, w2_ref, b2_ref, y_ref):
    x = x_ref[...]
    h = jnp.dot(x, w1_ref[...], preferred_element_type=jnp.float32)
    h = h + b1_ref[...]
    h = h * jax.nn.sigmoid(h)
    y = jnp.dot(h, w2_ref[...], preferred_element_type=jnp.float32)
    y_ref[...] = y + b2_ref[...]


def _mlp(x, W1, b1, W2, b2):
    return pl.pallas_call(
        _mlp_body,
        grid=(GRID,),
        in_specs=[
            pl.BlockSpec((BLK, N_IN), lambda i: (i, 0)),
            pl.BlockSpec((N_IN, N_HID), lambda i: (0, 0)),
            pl.BlockSpec((1, N_HID), lambda i: (0, 0)),
            pl.BlockSpec((N_HID, 1), lambda i: (0, 0)),
            pl.BlockSpec((1, 1), lambda i: (0, 0)),
        ],
        out_specs=pl.BlockSpec((BLK, 1), lambda i: (i, 0)),
        out_shape=jax.ShapeDtypeStruct((N, 1), jnp.float32),
    )(x, W1, b1.reshape(1, N_HID), W2, b2.reshape(1, 1))


def _seg_body(y_hbm, idx_hbm, out_hbm,
              idx_v, val_v, tidx_v, tval_v, acc_v, row_v, red_v, out_v,
              shared):
    wid = lax.axis_index("s")
    is_last = wid == NT - 1
    base = wid * CH
    pltpu.sync_copy(idx_hbm.at[pl.ds(base, CH)], idx_v)
    pltpu.sync_copy(y_hbm.at[pl.ds(base, CH)], val_v)
    pltpu.sync_copy(idx_hbm.at[pl.ds(NT * CH, TAIL)], tidx_v)
    pltpu.sync_copy(y_hbm.at[pl.ds(NT * CH, TAIL)], tval_v)

    lo = idx_v[pl.ds(0, 16)][0]
    hi = jnp.where(is_last, tidx_v[pl.ds(TAIL - 16, 16)][15],
                   idx_v[pl.ds(CH - 16, 16)][15])
    g0 = lax.shift_right_logical(lo, 4)
    g1 = lax.shift_right_logical(hi, 4)
    trips = g1 - g0 + 1

    zeros16 = jnp.zeros((16,), jnp.float32)

    def zrow_body(c, carry):
        row_v[pl.ds(c * 16, 16)] = zeros16
        return carry

    lax.fori_loop(0, M2 // 16, zrow_body, 0)

    def zero_body(c, carry):
        for r in range(NT):
            acc_v[pl.ds(r * SR + (g0 + c) * 16, 16)] = zeros16
        return carry

    lax.fori_loop(0, trips, zero_body, 0)

    lane_off = lax.iota(jnp.int32, 16) * SR

    def scat_body(i, carry):
        for u in range(6):
            iv = idx_v[pl.ds((i * 6 + u) * 16, 16)]
            vv = val_v[pl.ds((i * 6 + u) * 16, 16)]
            plsc.addupdate_scatter(acc_v, [iv + lane_off], vv)
        return carry

    lax.fori_loop(0, CH // 96, scat_body, 0)

    @pl.when(is_last)
    def _tail():
        def tscat_body(i, carry):
            iv = tidx_v[pl.ds(i * 16, 16)]
            vv = tval_v[pl.ds(i * 16, 16)]
            plsc.addupdate_scatter(acc_v, [iv + lane_off], vv)
            return carry

        lax.fori_loop(0, TAIL // 16, tscat_body, 0)

    def red_body(c, carry):
        s = acc_v[pl.ds((g0 + c) * 16, 16)]
        for r in range(1, NT):
            s = s + acc_v[pl.ds(r * SR + (g0 + c) * 16, 16)]
        row_v[pl.ds((g0 + c) * 16, 16)] = s
        return carry

    lax.fori_loop(0, trips, red_body, 0)

    pltpu.sync_copy(row_v, shared.at[wid])
    plsc.subcore_barrier()

    pltpu.sync_copy(shared.at[:, pl.ds(wid * 128, 128)], red_v)

    def fin_body(c, carry):
        s = red_v[0, pl.ds(c * 16, 16)]
        for r in range(1, NT):
            s = s + red_v[r, pl.ds(c * 16, 16)]
        out_v[pl.ds(c * 16, 16)] = s
        return carry

    lax.fori_loop(0, 128 // 16, fin_body, 0)

    @pl.when(jnp.logical_not(is_last))
    def _full_write():
        pltpu.sync_copy(out_v, out_hbm.at[pl.ds(wid * 128, 128)])

    @pl.when(is_last)
    def _last_write():
        pltpu.sync_copy(out_v.at[pl.ds(0, M - (NT - 1) * 128)],
                        out_hbm.at[pl.ds((NT - 1) * 128,
                                         M - (NT - 1) * 128)])


@functools.cache
def _make_seg():
    @functools.partial(
        pl.kernel,
        mesh=plsc.VectorSubcoreMesh(core_axis_name="c", subcore_axis_name="s",
                                    num_cores=1),
        out_type=jax.ShapeDtypeStruct((M,), jnp.float32),
        compiler_params=pltpu.CompilerParams(
            use_tc_tiling_on_sc=False, needs_layout_passes=False),
        scratch_types=[
            pltpu.VMEM((CH,), jnp.int32),
            pltpu.VMEM((CH,), jnp.float32),
            pltpu.VMEM((TAIL,), jnp.int32),
            pltpu.VMEM((TAIL,), jnp.float32),
            pltpu.VMEM((NT * SR,), jnp.float32),
            pltpu.VMEM((M2,), jnp.float32),
            pltpu.VMEM((NT, 128), jnp.float32),
            pltpu.VMEM((128,), jnp.float32),
            pltpu.VMEM_SHARED((NT, M2), jnp.float32),
        ],
    )
    def _seg(y_hbm, idx_hbm, out_hbm, *scratch):
        _seg_body(y_hbm, idx_hbm, out_hbm, *scratch)

    return _seg


def kernel(scalar_representation, idx_m, W1, b1, W2, b2):
    y = _mlp(scalar_representation, W1, b1, W2, b2).reshape(N)
    return _make_seg()(y, idx_m)


# speedup vs baseline: 1.1182x; 1.0059x over previous
import functools

import jax
import jax.numpy as jnp
from jax import lax
from jax.experimental import pallas as pl
from jax.experimental.pallas import tpu as pltpu
from jax.experimental.pallas import tpu_sc as plsc

N = 100000
N_IN = 128
N_HID = 32
M = 2000

M2 = 2048
SR = M2 + 1
NT = 16
CH = 6240
TAIL = N - NT * CH
BLK = 10000
GRID = N // BLK


def _mlp_body(x_ref, w1_ref, b1_ref, w2_ref, b2_ref, y_ref):
    x = x_ref[...]
    h = jnp.dot(x, w1_ref[...], preferred_element_type=jnp.float32)
    h = h + b1_ref[...]
    h = h * jax.nn.sigmoid(h)
    y = jnp.dot(h, w2_ref[...], preferred_element_type=jnp.float32)
    y_ref[...] = y + b2_ref[...]


def _mlp(x, W1, b1, W2, b2):
    return pl.pallas_call(
        _mlp_body,
        grid=(GRID,),
        in_specs=[
            pl.BlockSpec((BLK, N_IN), lambda i: (i, 0)),
            pl.BlockSpec((N_IN, N_HID), lambda i: (0, 0)),
            pl.BlockSpec((1, N_HID), lambda i: (0, 0)),
            pl.BlockSpec((N_HID, 1), lambda i: (0, 0)),
            pl.BlockSpec((1, 1), lambda i: (0, 0)),
        ],
        out_specs=pl.BlockSpec((BLK, 1), lambda i: (i, 0)),
        out_shape=jax.ShapeDtypeStruct((N, 1), jnp.float32),
    )(x, W1, b1.reshape(1, N_HID), W2, b2.reshape(1, 1))


def _seg_body(y_hbm, idx_hbm, out_hbm,
              idx_v, val_v, tidx_v, tval_v, acc_v, row_v, red_v, out_v,
              shared):
    wid = lax.axis_index("s")
    is_last = wid == NT - 1
    base = wid * CH
    pltpu.sync_copy(idx_hbm.at[pl.ds(base, CH)], idx_v)
    pltpu.sync_copy(y_hbm.at[pl.ds(base, CH)], val_v)
    pltpu.sync_copy(idx_hbm.at[pl.ds(NT * CH, TAIL)], tidx_v)
    pltpu.sync_copy(y_hbm.at[pl.ds(NT * CH, TAIL)], tval_v)

    lo = idx_v[pl.ds(0, 16)][0]
    hi = jnp.where(is_last, tidx_v[pl.ds(TAIL - 16, 16)][15],
                   idx_v[pl.ds(CH - 16, 16)][15])
    g0 = lax.shift_right_logical(lo, 4)
    g1 = lax.shift_right_logical(hi, 4)
    trips = g1 - g0 + 1

    zeros16 = jnp.zeros((16,), jnp.float32)

    def zrow_body(c, carry):
        for u in range(8):
            row_v[pl.ds((c * 8 + u) * 16, 16)] = zeros16
        return carry

    lax.fori_loop(0, M2 // 128, zrow_body, 0)

    def zero_body(c, carry):
        for r in range(NT):
            acc_v[pl.ds(r * SR + (g0 + c) * 16, 16)] = zeros16
        return carry

    lax.fori_loop(0, trips, zero_body, 0)

    lane_off = lax.iota(jnp.int32, 16) * SR

    def scat_body(i, carry):
        for u in range(10):
            iv = idx_v[pl.ds((i * 10 + u) * 16, 16)]
            vv = val_v[pl.ds((i * 10 + u) * 16, 16)]
            plsc.addupdate_scatter(acc_v, [iv + lane_off], vv)
        return carry

    lax.fori_loop(0, CH // 160, scat_body, 0)

    @pl.when(is_last)
    def _tail():
        def tscat_body(i, carry):
            iv = tidx_v[pl.ds(i * 16, 16)]
            vv = tval_v[pl.ds(i * 16, 16)]
            plsc.addupdate_scatter(acc_v, [iv + lane_off], vv)
            return carry

        lax.fori_loop(0, TAIL // 16, tscat_body, 0)

    def red_body(c, carry):
        s = acc_v[pl.ds((g0 + c) * 16, 16)]
        for r in range(1, NT):
            s = s + acc_v[pl.ds(r * SR + (g0 + c) * 16, 16)]
        row_v[pl.ds((g0 + c) * 16, 16)] = s
        return carry

    lax.fori_loop(0, trips, red_body, 0)

    pltpu.sync_copy(row_v, shared.at[wid])
    plsc.subcore_barrier()

    pltpu.sync_copy(shared.at[:, pl.ds(wid * 128, 128)], red_v)

    def fin_body(c, carry):
        s = red_v[0, pl.ds(c * 16, 16)]
        for r in range(1, NT):
            s = s + red_v[r, pl.ds(c * 16, 16)]
        out_v[pl.ds(c * 16, 16)] = s
        return carry

    lax.fori_loop(0, 128 // 16, fin_body, 0)

    @pl.when(jnp.logical_not(is_last))
    def _full_write():
        pltpu.sync_copy(out_v, out_hbm.at[pl.ds(wid * 128, 128)])

    @pl.when(is_last)
    def _last_write():
        pltpu.sync_copy(out_v.at[pl.ds(0, M - (NT - 1) * 128)],
                        out_hbm.at[pl.ds((NT - 1) * 128,
                                         M - (NT - 1) * 128)])


@functools.cache
def _make_seg():
    @functools.partial(
        pl.kernel,
        mesh=plsc.VectorSubcoreMesh(core_axis_name="c", subcore_axis_name="s",
                                    num_cores=1),
        out_type=jax.ShapeDtypeStruct((M,), jnp.float32),
        compiler_params=pltpu.CompilerParams(
            use_tc_tiling_on_sc=False, needs_layout_passes=False),
        scratch_types=[
            pltpu.VMEM((CH,), jnp.int32),
            pltpu.VMEM((CH,), jnp.float32),
            pltpu.VMEM((TAIL,), jnp.int32),
            pltpu.VMEM((TAIL,), jnp.float32),
            pltpu.VMEM((NT * SR,), jnp.float32),
            pltpu.VMEM((M2,), jnp.float32),
            pltpu.VMEM((NT, 128), jnp.float32),
            pltpu.VMEM((128,), jnp.float32),
            pltpu.VMEM_SHARED((NT, M2), jnp.float32),
        ],
    )
    def _seg(y_hbm, idx_hbm, out_hbm, *scratch):
        _seg_body(y_hbm, idx_hbm, out_hbm, *scratch)

    return _seg


def kernel(scalar_representation, idx_m, W1, b1, W2, b2):
    y = _mlp(scalar_representation, W1, b1, W2, b2).reshape(N)
    return _make_seg()(y, idx_m)
